# Initial kernel scaffold; baseline (speedup 1.0000x reference)
import jax
import jax.numpy as jnp
from jax.experimental import pallas as pl


def kernel(tokens, pt_table, clip_table):
    raise NotImplementedError("write your pallas kernel here")


# speedup vs baseline: 7.6023x; 7.6023x over previous
import functools

import jax
import jax.numpy as jnp
from jax import lax
from jax.experimental import pallas as pl
from jax.experimental.pallas import tpu as pltpu
from jax.experimental.pallas import tpu_sc as plsc

_CTX = 77
_PROMPT = 10
_D = 512


def _build(batch):
    info = plsc.get_sparse_core_info()
    nc, ns = info.num_cores, info.num_subcores
    nw = nc * ns
    assert batch % nw == 0
    bpw = batch // nw
    mesh = plsc.VectorSubcoreMesh(core_axis_name="c", subcore_axis_name="s")

    n_tail = _CTX - 1 - _PROMPT

    @functools.partial(
        pl.kernel,
        mesh=mesh,
        out_type=jax.ShapeDtypeStruct((batch, _CTX, _D), jnp.float32),
        compiler_params=pltpu.CompilerParams(use_tc_tiling_on_sc=False),
        scratch_types=[
            pltpu.VMEM((bpw, 1), jnp.int32),
            pltpu.VMEM((bpw, _PROMPT), jnp.int32),
            pltpu.VMEM((bpw, n_tail), jnp.int32),
            pltpu.VMEM((_CTX, _D), jnp.float32),
            pltpu.VMEM((_CTX, _D), jnp.float32),
            pltpu.SemaphoreType.DMA,
            pltpu.SemaphoreType.DMA,
            pltpu.SemaphoreType.DMA,
            pltpu.SemaphoreType.DMA,
        ],
    )
    def k(idx0_hbm, idx1_hbm, idx2_hbm, pt_hbm, clip_hbm, out_hbm,
          idx0_v, idx1_v, idx2_v, buf_a, buf_b, gs_a, gs_b, os_a, os_b):
        wid = lax.axis_index("s") * nc + lax.axis_index("c")
        base = wid * bpw
        pltpu.sync_copy(idx0_hbm.at[pl.ds(base, bpw)], idx0_v)
        pltpu.sync_copy(idx1_hbm.at[pl.ds(base, bpw)], idx1_v)
        pltpu.sync_copy(idx2_hbm.at[pl.ds(base, bpw)], idx2_v)

        def fire_gathers(b, buf, sem):
            pltpu.async_copy(
                clip_hbm.at[idx0_v.at[b]], buf.at[pl.ds(0, 1)], sem)
            pltpu.async_copy(
                pt_hbm.at[idx1_v.at[b]], buf.at[pl.ds(1, _PROMPT)], sem)
            pltpu.async_copy(
                clip_hbm.at[idx2_v.at[b]], buf.at[pl.ds(1 + _PROMPT, n_tail)],
                sem)

        def wait_gathers(buf, sem):
            pltpu.make_async_copy(clip_hbm.at[pl.ds(0, _CTX)], buf, sem).wait()

        def fire_out(buf, b, sem):
            pltpu.async_copy(buf, out_hbm.at[base + b], sem)

        def wait_out(buf, sem):
            pltpu.make_async_copy(buf, out_hbm.at[base], sem).wait()

        fire_gathers(0, buf_a, gs_a)
        fire_gathers(1, buf_b, gs_b)

        def step(it, carry):
            b0 = 2 * it
            wait_gathers(buf_a, gs_a)
            fire_out(buf_a, b0, os_a)
            wait_gathers(buf_b, gs_b)
            fire_out(buf_b, b0 + 1, os_b)

            @pl.when(it < bpw // 2 - 1)
            def _():
                wait_out(buf_a, os_a)
                fire_gathers(b0 + 2, buf_a, gs_a)
                wait_out(buf_b, os_b)
                fire_gathers(b0 + 3, buf_b, gs_b)

            return carry

        lax.fori_loop(0, bpw // 2, step, 0)
        wait_out(buf_a, os_a)
        wait_out(buf_b, os_b)

    return k


def kernel(tokens, pt_table, clip_table):
    batch = tokens.shape[0]
    idx0 = tokens[:, 0:1].astype(jnp.int32)
    idx1 = tokens[:, 1:1 + _PROMPT].astype(jnp.int32)
    idx2 = tokens[:, 1 + _PROMPT:_CTX].astype(jnp.int32)
    return _build(batch)(idx0, idx1, idx2, pt_table, clip_table)


# speedup vs baseline: 7.6529x; 1.0067x over previous
import functools

import jax
import jax.numpy as jnp
from jax import lax
from jax.experimental import pallas as pl
from jax.experimental.pallas import tpu as pltpu
from jax.experimental.pallas import tpu_sc as plsc

_CTX = 77
_PROMPT = 10
_D = 512


def _build(batch):
    info = plsc.get_sparse_core_info()
    nc, ns = info.num_cores, info.num_subcores
    nw = nc * ns
    assert batch % nw == 0
    bpw = batch // nw
    assert bpw % 3 == 2 and bpw >= 5
    mesh = plsc.VectorSubcoreMesh(core_axis_name="c", subcore_axis_name="s")

    n_tail = _CTX - 1 - _PROMPT

    @functools.partial(
        pl.kernel,
        mesh=mesh,
        out_type=jax.ShapeDtypeStruct((batch, _CTX, _D), jnp.float32),
        compiler_params=pltpu.CompilerParams(use_tc_tiling_on_sc=False),
        scratch_types=[
            pltpu.VMEM((bpw, 1), jnp.int32),
            pltpu.VMEM((bpw, _PROMPT), jnp.int32),
            pltpu.VMEM((bpw, n_tail), jnp.int32),
            [pltpu.VMEM((_CTX, _D), jnp.float32)] * 3,
            [pltpu.SemaphoreType.DMA] * 3,
            [pltpu.SemaphoreType.DMA] * 3,
        ],
    )
    def k(idx0_hbm, idx1_hbm, idx2_hbm, pt_hbm, clip_hbm, out_hbm,
          idx0_v, idx1_v, idx2_v, bufs, gsems, osems):
        wid = lax.axis_index("s") * nc + lax.axis_index("c")
        base = wid * bpw
        pltpu.sync_copy(idx0_hbm.at[pl.ds(base, bpw)], idx0_v)
        pltpu.sync_copy(idx1_hbm.at[pl.ds(base, bpw)], idx1_v)
        pltpu.sync_copy(idx2_hbm.at[pl.ds(base, bpw)], idx2_v)

        def fire_gathers(b, p):
            buf, sem = bufs[p], gsems[p]
            pltpu.async_copy(
                clip_hbm.at[idx0_v.at[b]], buf.at[pl.ds(0, 1)], sem)
            pltpu.async_copy(
                pt_hbm.at[idx1_v.at[b]], buf.at[pl.ds(1, _PROMPT)], sem)
            pltpu.async_copy(
                clip_hbm.at[idx2_v.at[b]], buf.at[pl.ds(1 + _PROMPT, n_tail)],
                sem)

        def wait_gathers(p):
            pltpu.make_async_copy(
                clip_hbm.at[pl.ds(0, _CTX)], bufs[p], gsems[p]).wait()

        def fire_out(b, p):
            pltpu.async_copy(bufs[p], out_hbm.at[base + b], osems[p])

        def wait_out(p):
            pltpu.make_async_copy(bufs[p], out_hbm.at[base], osems[p]).wait()

        fire_gathers(0, 0)

        def step(it, carry):
            j0 = 3 * it
            for o in range(3):
                j = j0 + o
                wait_gathers(o)
                fire_out(j, o)

                @pl.when(j >= 2)
                def _():
                    wait_out((o + 1) % 3)

                fire_gathers(j + 1, (o + 1) % 3)
            return carry

        lax.fori_loop(0, (bpw - 2) // 3, step, 0)

        p = (bpw - 2) % 3
        wait_gathers(p)
        fire_out(bpw - 2, p)
        wait_out((p + 1) % 3)
        fire_gathers(bpw - 1, (p + 1) % 3)
        wait_gathers((p + 1) % 3)
        fire_out(bpw - 1, (p + 1) % 3)
        wait_out((p + 2) % 3)
        wait_out(p)
        wait_out((p + 1) % 3)

    return k


def kernel(tokens, pt_table, clip_table):
    batch = tokens.shape[0]
    idx0 = tokens[:, 0:1].astype(jnp.int32)
    idx1 = tokens[:, 1:1 + _PROMPT].astype(jnp.int32)
    idx2 = tokens[:, 1 + _PROMPT:_CTX].astype(jnp.int32)
    return _build(batch)(idx0, idx1, idx2, pt_table, clip_table)
